# baseline (speedup 1.0000x reference)
import functools

import numpy as np
import jax
import jax.numpy as jnp
from jax import lax
from jax.experimental import pallas as pl
from jax.experimental.pallas import tpu as pltpu
from jax.experimental.pallas import tpu_sc as plsc

_NB = 4
_NPOINTS = 16
_CLIP = 20.0


def _perm_tables():
    a = np.zeros([_NPOINTS, _NB], dtype=np.int32)
    for i in range(_NPOINTS):
        a[i, :] = np.array(list(np.binary_repr(i, _NB)), dtype=np.int32)
    c0 = np.zeros([_NPOINTS // 2, _NB], np.int32)
    c1 = np.zeros([_NPOINTS // 2, _NB], np.int32)
    for i in range(_NB):
        c0[:, i] = np.where(a[:, i] == 0)[0]
        c1[:, i] = np.where(a[:, i] == 1)[0]
    return c0.reshape(-1), c1.reshape(-1)


_G0, _G1 = _perm_tables()
_NSHADOW = 4


@functools.cache
def _build_sc_kernel(B, K, S):
    info = plsc.get_sparse_core_info()
    NC, NS = info.num_cores, info.num_subcores
    NW = NC * NS
    L = 16
    assert B % (NW * L) == 0
    rows_w = B // NW
    groups = rows_w // L
    KS = K * S
    OUT_W = (_NPOINTS // 2) * _NB

    mesh = plsc.VectorSubcoreMesh(core_axis_name="c", subcore_axis_name="s")

    @functools.partial(
        pl.kernel,
        out_type=jax.ShapeDtypeStruct((B * OUT_W // 128, 128), jnp.float32),
        mesh=mesh,
        compiler_params=pltpu.CompilerParams(needs_layout_passes=False),
        scratch_types=[
            pltpu.VMEM((rows_w * KS // 128, 128), jnp.int32),
            pltpu.VMEM((rows_w * K // 128, 128), jnp.float32),
            pltpu.VMEM((rows_w * OUT_W // 128, 128), jnp.float32),
        ],
    )
    def sc_kernel(pi_hbm, d_hbm, out_hbm, pi_v, d_v, out_v):
        wid = lax.axis_index("s") * NC + lax.axis_index("c")
        pi_f, d_f, out_f = pi_hbm, d_hbm, out_hbm
        n_pi = rows_w * KS // 128
        n_d = rows_w * K // 128
        n_o = rows_w * OUT_W // 128
        pltpu.sync_copy(pi_f.at[pl.ds(wid * n_pi, n_pi)], pi_v)
        pltpu.sync_copy(d_f.at[pl.ds(wid * n_d, n_d)], d_v)

        iota = lax.iota(jnp.int32, L)
        zero = iota * 0
        one = zero + 1
        lane2 = iota * (KS // 128)
        i_k = iota * K
        i_ow = iota * OUT_W
        svecs = [zero + s for s in range(S)]
        ninf = (zero + np.int32(~np.float32(np.inf).view(np.int32))
                ).astype(jnp.uint32)

        def group_body(g, carry):
            def k_body(k, maccs):
                prow = lane2 + (g * (L * KS // 128) + (k >> 4))
                cbase = (k & 15) * S
                bits = None
                for s in range(S):
                    pival = plsc.load_gather(pi_v, [prow, svecs[s] + cbase])
                    bit = jnp.left_shift(one, pival)
                    bits = bit if bits is None else (bits | bit)
                fd = i_k + (g * (L * K) + k)
                dval = plsc.load_gather(d_v, [fd >> 7, fd & 127]) * 0.5
                nd = ~plsc.bitcast(dval, jnp.int32)
                new = []
                sh = bits << (31 - _NPOINTS + 1)
                for v in range(_NPOINTS - 1, -1, -1):
                    cand = nd & (sh >> 31)
                    new.append(jnp.maximum(maccs[_NPOINTS - 1 - v],
                                           cand.astype(jnp.uint32)))
                    if v:
                        sh = sh << 1
                return tuple(new)

            maccs = lax.fori_loop(0, K, k_body, (ninf,) * _NPOINTS,
                                  unroll=4)
            ms = [plsc.bitcast(~maccs[_NPOINTS - 1 - v].astype(jnp.int32),
                               jnp.float32)
                  for v in range(_NPOINTS)]

            for t in range(OUT_W):
                llr = jnp.clip(ms[_G0[t]] - ms[_G1[t]], -_CLIP, _CLIP)
                fo = i_ow + (g * (L * OUT_W) + t)
                plsc.store_scatter(out_v, [fo >> 7, fo & 127], llr)
            return carry

        lax.fori_loop(0, groups, group_body, 0)
        pltpu.sync_copy(out_v, out_f.at[pl.ds(wid * n_o, n_o)])

    return sc_kernel


def kernel(y, r, dists, path_inds, path_syms):
    B, K = dists.shape
    S = path_inds.shape[2]
    out2d = _build_sc_kernel(B, K, S)(
        path_inds.reshape(B * K * S // 128, 128),
        dists.reshape(B * K // 128, 128))
    return out2d.reshape(B, _NPOINTS // 2, _NB)
